# baseline (speedup 1.0000x reference)
import functools

import jax
import jax.numpy as jnp
from jax import lax
from jax.experimental import pallas as pl
from jax.experimental.pallas import tpu as pltpu
from jax.experimental.pallas import tpu_sc as plsc

SPELL_LEN = 9
HIDDEN = 128
ROW = SPELL_LEN * HIDDEN

NC = 2
NS = 16
NW = NC * NS
CH = 32


@functools.lru_cache(maxsize=None)
def _make(batch, vocab):
    per_tile = batch // NW
    nch = per_tile // CH
    mesh = plsc.VectorSubcoreMesh(core_axis_name="c", subcore_axis_name="s")

    @functools.partial(
        pl.kernel,
        mesh=mesh,
        out_type=jax.ShapeDtypeStruct((batch, ROW), jnp.float32),
        scratch_types=[
            pltpu.VMEM((nch, CH), jnp.int32),
            pltpu.VMEM((CH, ROW), jnp.float32),
            pltpu.SemaphoreType.DMA,
        ],
    )
    def k(rs_hbm, table_hbm, out_hbm, idx_v, buf_v, gsem):
        wid = lax.axis_index("s") * NC + lax.axis_index("c")
        pltpu.sync_copy(rs_hbm.at[pl.ds(wid * nch, nch)], idx_v)
        base = wid * per_tile
        for j in range(nch):
            pltpu.async_copy(table_hbm.at[idx_v.at[j]], buf_v, gsem).wait()
            pltpu.sync_copy(buf_v, out_hbm.at[pl.ds(base + j * CH, CH)])

    return k


def kernel(rs_tensor, embedding_relation):
    batch = rs_tensor.shape[0]
    vocab = embedding_relation.shape[0] // SPELL_LEN
    table2 = embedding_relation.reshape(vocab, ROW)
    rs2 = rs_tensor.reshape(batch // CH, CH)
    out2 = _make(batch, vocab)(rs2, table2)
    return out2.reshape(batch, SPELL_LEN, HIDDEN)


# speedup vs baseline: 1.0110x; 1.0110x over previous
import functools

import jax
import jax.numpy as jnp
from jax import lax
from jax.experimental import pallas as pl
from jax.experimental.pallas import tpu as pltpu
from jax.experimental.pallas import tpu_sc as plsc

SPELL_LEN = 9
HIDDEN = 128
ROW = SPELL_LEN * HIDDEN

NC = 2
NS = 16
NW = NC * NS
CH = 32
NBUF = 3


@functools.lru_cache(maxsize=None)
def _make(batch, vocab):
    per_tile = batch // NW
    nch = per_tile // CH
    mesh = plsc.VectorSubcoreMesh(core_axis_name="c", subcore_axis_name="s")

    @functools.partial(
        pl.kernel,
        mesh=mesh,
        out_type=jax.ShapeDtypeStruct((batch, ROW), jnp.float32),
        scratch_types=[
            pltpu.VMEM((nch, CH), jnp.int32),
        ]
        + [pltpu.VMEM((CH, ROW), jnp.float32) for _ in range(NBUF)]
        + [pltpu.SemaphoreType.DMA for _ in range(2 * NBUF)],
    )
    def k(rs_hbm, table_hbm, out_hbm, idx_v, *rest):
        bufs = rest[:NBUF]
        gsems = rest[NBUF : 2 * NBUF]
        wsems = rest[2 * NBUF :]
        wid = lax.axis_index("s") * NC + lax.axis_index("c")
        pltpu.sync_copy(rs_hbm.at[pl.ds(wid * nch, nch)], idx_v)
        base = wid * per_tile

        def gather(j, b):
            return pltpu.async_copy(table_hbm.at[idx_v.at[j]], bufs[b], gsems[b])

        def write(j, b):
            return pltpu.async_copy(
                bufs[b], out_hbm.at[pl.ds(base + j * CH, CH)], wsems[b]
            )

        gdesc, wdesc = {}, {}
        depth = min(NBUF - 1, nch)
        for j in range(depth):
            gdesc[j % NBUF] = gather(j, j % NBUF)
        for j in range(nch):
            b = j % NBUF
            gdesc[b].wait()
            wdesc[b] = write(j, b)
            f = j + depth
            if f < nch:
                bf = f % NBUF
                if bf in wdesc:
                    wdesc[bf].wait()
                gdesc[bf] = gather(f, bf)
        for j in range(max(nch - depth - 1, 0), nch):
            wdesc[j % NBUF].wait()

    return k


def kernel(rs_tensor, embedding_relation):
    batch = rs_tensor.shape[0]
    vocab = embedding_relation.shape[0] // SPELL_LEN
    table2 = embedding_relation.reshape(vocab, ROW)
    rs2 = rs_tensor.reshape(batch // CH, CH)
    out2 = _make(batch, vocab)(rs2, table2)
    return out2.reshape(batch, SPELL_LEN, HIDDEN)


# speedup vs baseline: 2.8455x; 2.8144x over previous
import functools

import jax
import jax.numpy as jnp
from jax import lax
from jax.experimental import pallas as pl
from jax.experimental.pallas import tpu as pltpu
from jax.experimental.pallas import tpu_sc as plsc

SPELL_LEN = 9
HIDDEN = 128

NC = 2
NS = 16
NW = NC * NS
CHB = 8
NBUF = 4


@functools.lru_cache(maxsize=None)
def _make(batch, rows):
    per_tile = batch // NW
    nch = per_tile // CHB
    rows_ch = CHB * SPELL_LEN
    mesh = plsc.VectorSubcoreMesh(core_axis_name="c", subcore_axis_name="s")

    @functools.partial(
        pl.kernel,
        mesh=mesh,
        out_type=jax.ShapeDtypeStruct((batch * SPELL_LEN, HIDDEN), jnp.float32),
        scratch_types=[
            pltpu.VMEM((per_tile,), jnp.int32),
            pltpu.VMEM((per_tile * SPELL_LEN,), jnp.int32),
        ]
        + [pltpu.VMEM((rows_ch, HIDDEN), jnp.float32) for _ in range(NBUF)]
        + [pltpu.SemaphoreType.DMA for _ in range(2 * NBUF)],
    )
    def k(rs_hbm, table_hbm, out_hbm, rs_v, idx_v, *rest):
        bufs = rest[:NBUF]
        gsems = rest[NBUF : 2 * NBUF]
        wsems = rest[2 * NBUF :]
        wid = lax.axis_index("s") * NC + lax.axis_index("c")
        base = wid * per_tile
        pltpu.sync_copy(rs_hbm.at[pl.ds(base, per_tile)], rs_v)

        iota = lax.broadcasted_iota(jnp.int32, (16,), 0)

        def expand(m):
            for k in range(SPELL_LEN):
                p = iota + k * 16
                e = lax.shift_right_logical(p * 57, 9)
                s = p - e * SPELL_LEN
                r_lin = rs_v[pl.ds(m * 16, 16)]
                r = lax.gather(
                    r_lin,
                    e[:, None],
                    lax.GatherDimensionNumbers(
                        offset_dims=(),
                        collapsed_slice_dims=(0,),
                        start_index_map=(0,),
                    ),
                    (1,),
                    mode=lax.GatherScatterMode.PROMISE_IN_BOUNDS,
                )
                idx_v[pl.ds((m * SPELL_LEN + k) * 16, 16)] = r * SPELL_LEN + s

        for _m in range(per_tile // 16):
            expand(_m)

        def gather(c, b):
            return pltpu.async_copy(
                table_hbm.at[idx_v.at[pl.ds(c * rows_ch, rows_ch)]],
                bufs[b],
                gsems[b],
            )

        def write(c, b):
            return pltpu.async_copy(
                bufs[b],
                out_hbm.at[pl.ds((base + c * CHB) * SPELL_LEN, rows_ch)],
                wsems[b],
            )

        gdesc, wdesc = {}, {}
        depth = min(NBUF - 1, nch)
        for c in range(depth):
            gdesc[c % NBUF] = gather(c, c % NBUF)
        for c in range(nch):
            b = c % NBUF
            gdesc[b].wait()
            wdesc[b] = write(c, b)
            f = c + depth
            if f < nch:
                bf = f % NBUF
                if bf in wdesc:
                    wdesc[bf].wait()
                gdesc[bf] = gather(f, bf)
        for c in range(max(nch - depth - 1, 0), nch):
            wdesc[c % NBUF].wait()

    return k


def kernel(rs_tensor, embedding_relation):
    batch = rs_tensor.shape[0]
    rows = embedding_relation.shape[0]
    out = _make(batch, rows)(rs_tensor, embedding_relation)
    return out.reshape(batch, SPELL_LEN, HIDDEN)


# speedup vs baseline: 4.4399x; 1.5603x over previous
import functools

import jax
import jax.numpy as jnp
from jax import lax
from jax.experimental import pallas as pl
from jax.experimental.pallas import tpu as pltpu
from jax.experimental.pallas import tpu_sc as plsc

SPELL_LEN = 9
HIDDEN = 128

NC = 2
NS = 16
NW = NC * NS
CHB = 8
NBUF = 4


@functools.lru_cache(maxsize=None)
def _make(batch, rows):
    per_tile = batch // NW
    nch = per_tile // CHB
    rows_ch = CHB * SPELL_LEN
    mesh = plsc.VectorSubcoreMesh(core_axis_name="c", subcore_axis_name="s")

    @functools.partial(
        pl.kernel,
        mesh=mesh,
        out_type=jax.ShapeDtypeStruct((batch, SPELL_LEN, HIDDEN), jnp.float32),
        scratch_types=[
            pltpu.VMEM((per_tile,), jnp.int32),
            pltpu.VMEM((per_tile * SPELL_LEN,), jnp.int32),
        ]
        + [pltpu.VMEM((rows_ch, HIDDEN), jnp.float32) for _ in range(NBUF)]
        + [pltpu.SemaphoreType.DMA for _ in range(2 * NBUF)],
    )
    def k(rs_hbm, table_hbm, out_hbm, rs_v, idx_v, *rest):
        bufs = rest[:NBUF]
        gsems = rest[NBUF : 2 * NBUF]
        wsems = rest[2 * NBUF :]
        wid = lax.axis_index("s") * NC + lax.axis_index("c")
        base = wid * per_tile
        pltpu.sync_copy(rs_hbm.at[pl.ds(base, per_tile)], rs_v)

        iota = lax.broadcasted_iota(jnp.int32, (16,), 0)

        def expand(m):
            for k in range(SPELL_LEN):
                p = iota + k * 16
                e = lax.shift_right_logical(p * 57, 9)
                s = p - e * SPELL_LEN
                r_lin = rs_v[pl.ds(m * 16, 16)]
                r = lax.gather(
                    r_lin,
                    e[:, None],
                    lax.GatherDimensionNumbers(
                        offset_dims=(),
                        collapsed_slice_dims=(0,),
                        start_index_map=(0,),
                    ),
                    (1,),
                    mode=lax.GatherScatterMode.PROMISE_IN_BOUNDS,
                )
                idx_v[pl.ds((m * SPELL_LEN + k) * 16, 16)] = r * SPELL_LEN + s

        for _m in range(per_tile // 16):
            expand(_m)

        def gather(c, b):
            return pltpu.async_copy(
                table_hbm.at[idx_v.at[pl.ds(c * rows_ch, rows_ch)]],
                bufs[b],
                gsems[b],
            )

        def write(c, b):
            descs = []
            for i in range(CHB):
                descs.append(
                    pltpu.async_copy(
                        bufs[b].at[pl.ds(i * SPELL_LEN, SPELL_LEN)],
                        out_hbm.at[base + c * CHB + i],
                        wsems[b],
                    )
                )
            return descs

        gdesc, wdesc = {}, {}
        depth = min(NBUF - 1, nch)
        for c in range(depth):
            gdesc[c % NBUF] = gather(c, c % NBUF)
        for c in range(nch):
            b = c % NBUF
            gdesc[b].wait()
            wdesc[b] = write(c, b)
            f = c + depth
            if f < nch:
                bf = f % NBUF
                if bf in wdesc:
                    for d in wdesc[bf]:
                        d.wait()
                gdesc[bf] = gather(f, bf)
        for c in range(max(nch - depth - 1, 0), nch):
            for d in wdesc[c % NBUF]:
                d.wait()

    return k


def kernel(rs_tensor, embedding_relation):
    batch = rs_tensor.shape[0]
    rows = embedding_relation.shape[0]
    return _make(batch, rows)(rs_tensor, embedding_relation)
